# baseline (speedup 1.0000x reference)
import jax
import jax.numpy as jnp
from jax.experimental import pallas as pl

B, L, N, C = 8, 12, 50000, 3
EDIM = 32

BLOCK_N = 1024
NUM_NB = -(-N // BLOCK_N)

_DN_T = (((1,), (1,)), ((), ()))


def _stid_kernel(x_ref, wr_ref, b_ref, node_ref, tidT_ref, diwT_ref, out_ref):
    ts = b_ref[...]
    for l in range(L):
        ts = ts + jax.lax.dot_general(
            wr_ref[l], x_ref[0, l], _DN_T, preferred_element_type=jnp.float32)

    tvals = jnp.transpose(x_ref[0, L - 1, :, 1:3])
    tid_idx = jnp.clip((tvals[0:1, :] * 288.0).astype(jnp.int32), 0, 287)
    diw_idx = jnp.clip((tvals[1:2, :] * 7.0).astype(jnp.int32), 0, 6)

    VW = 128
    c0 = tidT_ref[:, 0:VW]
    c1 = tidT_ref[:, VW:2 * VW]
    c2 = tidT_ref[:, 2 * VW:3 * VW]
    dw = diwT_ref[:, 0:VW]
    tid_parts = []
    diw_parts = []
    for j in range(BLOCK_N // VW):
        tj = jnp.broadcast_to(tid_idx[:, VW * j:VW * (j + 1)], (EDIM, VW))
        g0 = jnp.take_along_axis(c0, jnp.clip(tj, 0, VW - 1), axis=1)
        g1 = jnp.take_along_axis(c1, jnp.clip(tj - VW, 0, VW - 1), axis=1)
        g2 = jnp.take_along_axis(c2, jnp.clip(tj - 2 * VW, 0, VW - 1), axis=1)
        tid_parts.append(jnp.where(tj < VW, g0, jnp.where(tj < 2 * VW, g1, g2)))
        dj = jnp.broadcast_to(diw_idx[:, VW * j:VW * (j + 1)], (EDIM, VW))
        diw_parts.append(jnp.take_along_axis(dw, dj, axis=1))
    tid = jnp.concatenate(tid_parts, axis=1)
    diw = jnp.concatenate(diw_parts, axis=1)

    eye32 = jnp.eye(EDIM, dtype=jnp.float32)
    node = jax.lax.dot_general(eye32, node_ref[...], _DN_T,
                               preferred_element_type=jnp.float32)

    out_ref[0, 0:32, :] = ts
    out_ref[0, 32:64, :] = node
    out_ref[0, 64:96, :] = tid
    out_ref[0, 96:128, :] = diw


def kernel(x, node_emb, time_in_day_emb, day_in_week_emb, W, b):
    Wr = W.reshape(EDIM, L, C).transpose(1, 0, 2)
    tidT = jnp.pad(time_in_day_emb.T, ((0, 0), (0, 512 - 288)))
    diwT = jnp.pad(day_in_week_emb.T, ((0, 0), (0, 256 - 7)))
    b2 = b.reshape(EDIM, 1)

    out = pl.pallas_call(
        _stid_kernel,
        grid=(NUM_NB, B),
        in_specs=[
            pl.BlockSpec((1, L, BLOCK_N, C), lambda nb, bb: (bb, 0, nb, 0)),
            pl.BlockSpec((L, EDIM, C), lambda nb, bb: (0, 0, 0)),
            pl.BlockSpec((EDIM, 1), lambda nb, bb: (0, 0)),
            pl.BlockSpec((BLOCK_N, EDIM), lambda nb, bb: (nb, 0)),
            pl.BlockSpec((EDIM, 512), lambda nb, bb: (0, 0)),
            pl.BlockSpec((EDIM, 256), lambda nb, bb: (0, 0)),
        ],
        out_specs=pl.BlockSpec((1, 4 * EDIM, BLOCK_N), lambda nb, bb: (bb, 0, nb)),
        out_shape=jax.ShapeDtypeStruct((B, 4 * EDIM, N), jnp.float32),
    )(x, Wr, b2, node_emb, tidT, diwT)
    return out[..., None]


# speedup vs baseline: 5.4268x; 5.4268x over previous
import jax
import jax.numpy as jnp
from jax.experimental import pallas as pl

B, L, N, C = 8, 12, 50000, 3
EDIM = 32
LC = L * C

BLOCK_N = 2048
NUM_NB = -(-N // BLOCK_N)


def _stid_kernel(xt_ref, w_ref, b_ref, node_ref, tidT_ref, diwT_ref, out_ref):
    xt = xt_ref[0]
    ts = jax.lax.dot_general(
        w_ref[...], xt, (((1,), (0,)), ((), ())),
        preferred_element_type=jnp.float32)
    ts = ts + b_ref[...]

    tid_idx = jnp.clip((xt[34:35, :] * 288.0).astype(jnp.int32), 0, 287)
    diw_idx = jnp.clip((xt[35:36, :] * 7.0).astype(jnp.int32), 0, 6)

    VW = 128
    c0 = tidT_ref[:, 0:VW]
    c1 = tidT_ref[:, VW:2 * VW]
    c2 = tidT_ref[:, 2 * VW:3 * VW]
    dw = diwT_ref[...]
    tid_parts = []
    diw_parts = []
    for j in range(BLOCK_N // VW):
        tj = jnp.broadcast_to(tid_idx[:, VW * j:VW * (j + 1)], (EDIM, VW))
        g0 = jnp.take_along_axis(c0, jnp.clip(tj, 0, VW - 1), axis=1)
        g1 = jnp.take_along_axis(c1, jnp.clip(tj - VW, 0, VW - 1), axis=1)
        g2 = jnp.take_along_axis(c2, jnp.clip(tj - 2 * VW, 0, VW - 1), axis=1)
        tid_parts.append(jnp.where(tj < VW, g0, jnp.where(tj < 2 * VW, g1, g2)))
        dj = jnp.broadcast_to(diw_idx[:, VW * j:VW * (j + 1)], (EDIM, VW))
        diw_parts.append(jnp.take_along_axis(dw, dj, axis=1))
    tid = jnp.concatenate(tid_parts, axis=1)
    diw = jnp.concatenate(diw_parts, axis=1)

    out_ref[0, 0:32, 0, :] = ts
    out_ref[0, 32:64, 0, :] = node_ref[...]
    out_ref[0, 64:96, 0, :] = tid
    out_ref[0, 96:128, 0, :] = diw


def kernel(x, node_emb, time_in_day_emb, day_in_week_emb, W, b):
    xt = jnp.transpose(x, (0, 1, 3, 2)).reshape(B, LC, N)
    nodeT = node_emb.T
    tidT = jnp.pad(time_in_day_emb.T, ((0, 0), (0, 384 - 288)))
    diwT = jnp.pad(day_in_week_emb.T, ((0, 0), (0, 128 - 7)))
    b2 = b.reshape(EDIM, 1)

    out = pl.pallas_call(
        _stid_kernel,
        grid=(NUM_NB, B),
        in_specs=[
            pl.BlockSpec((1, LC, BLOCK_N), lambda nb, bb: (bb, 0, nb)),
            pl.BlockSpec((EDIM, LC), lambda nb, bb: (0, 0)),
            pl.BlockSpec((EDIM, 1), lambda nb, bb: (0, 0)),
            pl.BlockSpec((EDIM, BLOCK_N), lambda nb, bb: (0, nb)),
            pl.BlockSpec((EDIM, 384), lambda nb, bb: (0, 0)),
            pl.BlockSpec((EDIM, 128), lambda nb, bb: (0, 0)),
        ],
        out_specs=pl.BlockSpec((1, 4 * EDIM, 1, BLOCK_N),
                               lambda nb, bb: (bb, 0, 0, nb)),
        out_shape=jax.ShapeDtypeStruct((B, 4 * EDIM, 1, N), jnp.float32),
    )(xt, W, b2, nodeT, tidT, diwT)
    return out.reshape(B, 4 * EDIM, N, 1)
